# baseline (speedup 1.0000x reference)
import functools

import jax
import jax.numpy as jnp
from jax import lax
from jax.experimental import pallas as pl
from jax.experimental.pallas import tpu as pltpu
from jax.experimental.pallas import tpu_sc as plsc

LANES = 16


@functools.lru_cache(maxsize=None)
def _build_sc_pool_gather(B, HIST, E, n_items_pad, n_users):
    info = plsc.get_sparse_core_info()
    NC, NS = info.num_cores, info.num_subcores
    NW = NC * NS
    BPW = B // NW
    HALF = HIST // 2
    assert B % NW == 0 and HIST % 2 == 0 and HALF <= 128 and E % LANES == 0
    NV = E // LANES

    mesh = plsc.VectorSubcoreMesh(core_axis_name="c", subcore_axis_name="s")
    f32 = jnp.float32

    @functools.partial(
        pl.kernel,
        out_type=(
            jax.ShapeDtypeStruct((B, E), f32),
            jax.ShapeDtypeStruct((B, E), f32),
            jax.ShapeDtypeStruct((B, E), f32),
        ),
        mesh=mesh,
        compiler_params=pltpu.CompilerParams(use_tc_tiling_on_sc=False),
        scratch_types=[
            pltpu.VMEM((2 * BPW, HALF), jnp.int32),
            pltpu.VMEM((BPW,), jnp.int32),
            pltpu.VMEM((BPW,), jnp.int32),
            pltpu.VMEM((HIST, E), f32),
            pltpu.VMEM((HIST, E), f32),
            pltpu.VMEM((BPW, E), f32),
            pltpu.VMEM((BPW, E), f32),
            pltpu.VMEM((BPW, E), f32),
            pltpu.SemaphoreType.DMA,
            pltpu.SemaphoreType.DMA,
            pltpu.SemaphoreType.DMA,
            pltpu.SemaphoreType.DMA,
        ],
    )
    def sc_kernel(seq_hbm, tgt_hbm, usr_hbm, item_hbm, user_hbm,
                  pool_out, tgt_out, usr_out,
                  seq_v, tgti_v, usri_v, rows0, rows1, pool_v, trows, urows,
                  sem0, sem1, semt, semu):
        wid = lax.axis_index("s") * NC + lax.axis_index("c")
        base = wid * BPW

        pltpu.sync_copy(seq_hbm.at[pl.ds(2 * base, 2 * BPW)], seq_v)
        pltpu.sync_copy(tgt_hbm.at[pl.ds(base, BPW)], tgti_v)
        pltpu.sync_copy(usr_hbm.at[pl.ds(base, BPW)], usri_v)

        tcopy = pltpu.async_copy(item_hbm.at[tgti_v], trows, semt)
        ucopy = pltpu.async_copy(user_hbm.at[usri_v], urows, semu)

        rows = (rows0, rows1)
        sems = (sem0, sem1)

        def issue(s, b):
            pltpu.async_copy(item_hbm.at[seq_v.at[2 * s]],
                             rows[b].at[pl.ds(0, HALF)], sems[b])
            pltpu.async_copy(item_hbm.at[seq_v.at[2 * s + 1]],
                             rows[b].at[pl.ds(HALF, HALF)], sems[b])

        def wait(s, b):
            pltpu.make_async_copy(item_hbm.at[seq_v.at[2 * s]],
                                  rows[b].at[pl.ds(0, HALF)], sems[b]).wait()
            pltpu.make_async_copy(item_hbm.at[seq_v.at[2 * s + 1]],
                                  rows[b].at[pl.ds(HALF, HALF)], sems[b]).wait()

        for b in range(2):
            issue(b, b)

        inv = f32(1.0 / HIST)
        zeros = (jnp.zeros((LANES,), f32),) * NV

        @pl.loop(0, BPW, step=2)
        def _(s0):
            for b in range(2):
                s = s0 + b
                wait(s, b)
                r = rows[b]

                @pl.loop(0, HIST, init_carry=zeros, unroll=8)
                def acc(j, carry):
                    return tuple(carry[k] + r[j, pl.ds(k * LANES, LANES)]
                                 for k in range(NV))

                for k in range(NV):
                    pool_v[s, pl.ds(k * LANES, LANES)] = acc[k] * inv

                @pl.when(s + 2 < BPW)
                def _():
                    issue(s + 2, b)

        tcopy.wait()
        ucopy.wait()
        pltpu.sync_copy(pool_v, pool_out.at[pl.ds(base, BPW)])
        pltpu.sync_copy(trows, tgt_out.at[pl.ds(base, BPW)])
        pltpu.sync_copy(urows, usr_out.at[pl.ds(base, BPW)])

    return sc_kernel


def _mlp_body(p_ref, t_ref, u_ref, w1_ref, b1_ref, w2_ref, b2_ref, o_ref):
    E = p_ref.shape[1]
    dn = (((1,), (1,)), ((), ()))
    h = (lax.dot_general(p_ref[...], w1_ref[:, 0:E], dn,
                         preferred_element_type=jnp.float32)
         + lax.dot_general(t_ref[...], w1_ref[:, E:2 * E], dn,
                           preferred_element_type=jnp.float32)
         + lax.dot_general(u_ref[...], w1_ref[:, 2 * E:3 * E], dn,
                           preferred_element_type=jnp.float32)
         + b1_ref[...])
    h = jnp.maximum(h, 0.0)
    o_ref[...] = jnp.sum(h * w2_ref[...], axis=1, keepdims=True) + b2_ref[...]


def kernel(user_ids, input_seq, target_item, item_emb, user_emb, W1, b1, W2, b2):
    B, HIST = input_seq.shape
    E = item_emb.shape[1]
    pad_idx = item_emb.shape[0] - 1

    seq = jnp.where(input_seq == -1, pad_idx, input_seq).astype(jnp.int32)
    tgt = jnp.where(target_item == -1, pad_idx, target_item).astype(jnp.int32)
    usr = user_ids.astype(jnp.int32)
    seq2 = seq.reshape(2 * B, HIST // 2)

    sc = _build_sc_pool_gather(B, HIST, E, item_emb.shape[0], user_emb.shape[0])
    pooled, tgt_rows, usr_rows = sc(seq2, tgt, usr, item_emb, user_emb)

    out = pl.pallas_call(
        _mlp_body,
        out_shape=jax.ShapeDtypeStruct((B, 1), jnp.float32),
    )(pooled, tgt_rows, usr_rows, W1, b1.reshape(1, E), W2, b2.reshape(1, 1))
    return out


# speedup vs baseline: 1.6033x; 1.6033x over previous
import functools

import jax
import jax.numpy as jnp
from jax import lax
from jax.experimental import pallas as pl
from jax.experimental.pallas import tpu as pltpu
from jax.experimental.pallas import tpu_sc as plsc

LANES = 16


@functools.lru_cache(maxsize=None)
def _build_sc_pool_gather(B, HIST, E, n_items_pad, n_users):
    info = plsc.get_sparse_core_info()
    NC, NS = info.num_cores, info.num_subcores
    NW = NC * NS
    BPW = B // NW
    assert B % NW == 0 and E % LANES == 0
    NV = E // LANES

    mesh = plsc.VectorSubcoreMesh(core_axis_name="c", subcore_axis_name="s")
    f32 = jnp.float32

    @functools.partial(
        pl.kernel,
        out_type=(
            jax.ShapeDtypeStruct((B, E), f32),
            jax.ShapeDtypeStruct((B, E), f32),
            jax.ShapeDtypeStruct((B, E), f32),
        ),
        mesh=mesh,
        compiler_params=pltpu.CompilerParams(use_tc_tiling_on_sc=True),
        scratch_types=[
            pltpu.VMEM((2, HIST), jnp.int32),
            pltpu.VMEM((BPW,), jnp.int32),
            pltpu.VMEM((BPW,), jnp.int32),
            pltpu.VMEM((HIST, E), f32),
            pltpu.VMEM((HIST, E), f32),
            pltpu.VMEM((BPW, E), f32),
            pltpu.VMEM((BPW, E), f32),
            pltpu.VMEM((BPW, E), f32),
            pltpu.SemaphoreType.DMA,
            pltpu.SemaphoreType.DMA,
            pltpu.SemaphoreType.DMA,
            pltpu.SemaphoreType.DMA,
        ],
    )
    def sc_kernel(seq_hbm, tgt_hbm, usr_hbm, item_hbm, user_hbm,
                  pool_out, tgt_out, usr_out,
                  seq_v, tgti_v, usri_v,
                  rows0, rows1, pool_v, trows, urows,
                  sem0, sem1, semt, semu):
        wid = lax.axis_index("s") * NC + lax.axis_index("c")
        base = wid * BPW

        pltpu.sync_copy(tgt_hbm.at[pl.ds(base, BPW)], tgti_v)
        pltpu.sync_copy(usr_hbm.at[pl.ds(base, BPW)], usri_v)

        @pl.loop(0, BPW // LANES)
        def _(c):
            vt = tgti_v[pl.ds(c * LANES, LANES)]
            vu = usri_v[pl.ds(c * LANES, LANES)]
            for l in range(LANES):
                pltpu.async_copy(item_hbm.at[pl.ds(vt[l], 1)],
                                 trows.at[pl.ds(c * LANES + l, 1)], semt)
                pltpu.async_copy(user_hbm.at[pl.ds(vu[l], 1)],
                                 urows.at[pl.ds(c * LANES + l, 1)], semu)

        rows = (rows0, rows1)
        sems = (sem0, sem1)

        def stage_idx(s, b):
            pltpu.sync_copy(seq_hbm.at[pl.ds(base + s, 1)],
                            seq_v.at[pl.ds(b, 1)])

        NFULL = HIST // LANES
        NTAIL = HIST - NFULL * LANES

        def issue(b):
            @pl.loop(0, NFULL)
            def _(c):
                v = seq_v[b, pl.ds(c * LANES, LANES)]
                for l in range(LANES):
                    pltpu.async_copy(item_hbm.at[pl.ds(v[l], 1)],
                                     rows[b].at[pl.ds(c * LANES + l, 1)],
                                     sems[b])
            if NTAIL:
                v = seq_v[b, pl.ds(HIST - LANES, LANES)]
                for l in range(LANES - NTAIL, LANES):
                    pltpu.async_copy(
                        item_hbm.at[pl.ds(v[l], 1)],
                        rows[b].at[pl.ds(HIST - LANES + l, 1)], sems[b])

        def drain(b):
            @pl.loop(0, HIST, unroll=8)
            def _(j):
                pltpu.make_async_copy(item_hbm.at[pl.ds(0, 1)],
                                      rows[b].at[pl.ds(j, 1)], sems[b]).wait()

        for b in range(2):
            stage_idx(b, b)
            issue(b)

        inv = f32(1.0 / HIST)
        zeros = (jnp.zeros((LANES,), f32),) * NV

        @pl.loop(0, BPW, step=2)
        def _(s0):
            for b in range(2):
                s = s0 + b
                drain(b)
                r = rows[b]

                @pl.loop(0, HIST, init_carry=zeros, unroll=8)
                def acc(j, carry):
                    return tuple(carry[k] + r[j, pl.ds(k * LANES, LANES)]
                                 for k in range(NV))

                for k in range(NV):
                    pool_v[s, pl.ds(k * LANES, LANES)] = acc[k] * inv

                @pl.when(s + 2 < BPW)
                def _():
                    stage_idx(s + 2, b)
                    issue(b)

        @pl.loop(0, BPW, unroll=8)
        def _(i):
            pltpu.make_async_copy(item_hbm.at[pl.ds(0, 1)],
                                  trows.at[pl.ds(i, 1)], semt).wait()
            pltpu.make_async_copy(user_hbm.at[pl.ds(0, 1)],
                                  urows.at[pl.ds(i, 1)], semu).wait()

        pltpu.sync_copy(pool_v, pool_out.at[pl.ds(base, BPW)])
        pltpu.sync_copy(trows, tgt_out.at[pl.ds(base, BPW)])
        pltpu.sync_copy(urows, usr_out.at[pl.ds(base, BPW)])

    return sc_kernel


def _mlp_body(p_ref, t_ref, u_ref, w1_ref, b1_ref, w2_ref, b2_ref, o_ref):
    E = p_ref.shape[1]
    dn = (((1,), (1,)), ((), ()))
    h = (lax.dot_general(p_ref[...], w1_ref[:, 0:E], dn,
                         preferred_element_type=jnp.float32)
         + lax.dot_general(t_ref[...], w1_ref[:, E:2 * E], dn,
                           preferred_element_type=jnp.float32)
         + lax.dot_general(u_ref[...], w1_ref[:, 2 * E:3 * E], dn,
                           preferred_element_type=jnp.float32)
         + b1_ref[...])
    h = jnp.maximum(h, 0.0)
    o_ref[...] = jnp.sum(h * w2_ref[...], axis=1, keepdims=True) + b2_ref[...]


def kernel(user_ids, input_seq, target_item, item_emb, user_emb, W1, b1, W2, b2):
    B, HIST = input_seq.shape
    E = item_emb.shape[1]
    pad_idx = item_emb.shape[0] - 1

    seq = jnp.where(input_seq == -1, pad_idx, input_seq).astype(jnp.int32)
    tgt = jnp.where(target_item == -1, pad_idx, target_item).astype(jnp.int32)
    usr = user_ids.astype(jnp.int32)

    sc = _build_sc_pool_gather(B, HIST, E, item_emb.shape[0], user_emb.shape[0])
    pooled, tgt_rows, usr_rows = sc(seq, tgt, usr, item_emb, user_emb)

    out = pl.pallas_call(
        _mlp_body,
        out_shape=jax.ShapeDtypeStruct((B, 1), jnp.float32),
    )(pooled, tgt_rows, usr_rows, W1, b1.reshape(1, E), W2, b2.reshape(1, 1))
    return out
